# baseline (speedup 1.0000x reference)
import jax
import jax.numpy as jnp
from jax import lax
from jax.experimental import pallas as pl
from jax.experimental.pallas import tpu as pltpu
from jax.experimental.pallas import tpu_sc as plsc

N_ROWS = 10000
F = 256
H = 128
E = 160000
NC = 2
NS = 16
L = 16
K = 80
NB = 4
EP = 163840
PER_SUB = EP // NS
NCH = PER_SUB // K
NCHT = EP // K
RD = 80
NRD = N_ROWS // RD
RDPS = -(-NRD // NS)


def _body(pk_ref, di_ref, dj_ref, ustk_ref, istk_ref,
          out_u_ref, out_i_ref,
          pk_s, rows_s, dv, acc_sh, sem_m, sem_g, sem_s):
    cid = lax.axis_index("c")
    sid = lax.axis_index("s")
    goff = cid * N_ROWS
    g0 = sid * NCH
    zero16 = jnp.zeros((L,), jnp.float32)

    def _splat(vec, lane):
        return vec.at[jnp.full((L,), lane, dtype=jnp.int32)].get(
            mode="promise_in_bounds")

    for d in range(2):
        gc, sc = (0, 1) if d == 0 else (1, 0)
        table_ref = istk_ref if d == 0 else ustk_ref
        own_ref = ustk_ref if d == 0 else istk_ref
        dd_ref = di_ref if d == 0 else dj_ref
        out_ref = out_u_ref if d == 0 else out_i_ref

        def _fix(pk):
            for g in range(K // L):
                sl = pl.ds(g * L, L)
                pk[gc, sl] = pk[gc, sl] + goff

        def _scale(rows, pk):
            def egroup(g, _):
                vg = lax.bitcast_convert_type(
                    pk[2, pl.ds(g * L, L)], jnp.float32)

                def erow(i, _):
                    vsp = _splat(vg, i)
                    e = g * L + i
                    for f in range(H // L):
                        sl = pl.ds(f * L, L)
                        rows[e, sl] = rows[e, sl] * vsp
                    return 0

                lax.fori_loop(0, L, erow, 0, unroll=2)
                return 0

            lax.fori_loop(0, K // L, egroup, 0)

        def zrow(r, _):
            for f in range(H // L):
                rows_s[0][r, pl.ds(f * L, L)] = zero16
            return 0

        lax.fori_loop(0, RD, zrow, 0)
        for j in range(RDPS):
            ci = sid + j * NS

            @pl.when(ci < NRD)
            def _():
                pltpu.sync_copy(rows_s[0].at[pl.ds(0, RD)],
                                acc_sh.at[pl.ds(ci * RD, RD)])
        plsc.subcore_barrier()

        def _meta_start(c, s):
            pltpu.async_copy(pk_ref.at[g0 + c], pk_s[s], sem_m[s])

        def _meta_wait(c, s):
            pltpu.make_async_copy(pk_ref.at[g0 + c], pk_s[s],
                                  sem_m[s]).wait()

        def _gather_start(s):
            pltpu.async_copy(table_ref.at[pk_s[s].at[gc]], rows_s[s],
                             sem_g[s])

        def _gather_wait(s):
            pltpu.make_async_copy(table_ref.at[pk_s[s].at[gc]], rows_s[s],
                                  sem_g[s]).wait()

        def _scat_start(s):
            pltpu.async_copy(rows_s[s], acc_sh.at[pk_s[s].at[sc]],
                             sem_s[s], add=True)

        def _scat_wait(s):
            pltpu.make_async_copy(rows_s[s], acc_sh.at[pk_s[s].at[sc]],
                                  sem_s[s]).wait()


        def quad(t, _):
            for j in range(NB):
                c = NB * t + j
                s = j

                @pl.when(c + 1 < NCH)
                def _():
                    _meta_wait(c + 1, (s + 1) % NB)
                    _fix(pk_s[(s + 1) % NB])

                @pl.when(c >= 2)
                def _():
                    _scat_wait((s + 2) % NB)

                @pl.when(c + 2 < NCH)
                def _():
                    _meta_start(c + 2, (s + 2) % NB)
                _gather_wait(s)

                @pl.when(c + 1 < NCH)
                def _():
                    _gather_start((s + 1) % NB)
                _scale(rows_s[s], pk_s[s])
                _scat_start(s)
            return 0

        _meta_start(0, 0)
        _meta_start(1, 1)
        _meta_wait(0, 0)
        _fix(pk_s[0])
        _gather_start(0)
        lax.fori_loop(0, NCH // NB, quad, 0)
        _scat_wait((NCH - 2) % NB)
        _scat_wait((NCH - 1) % NB)
        plsc.subcore_barrier()

        for j in range(RDPS):
            ci = sid + j * NS

            @pl.when(ci < NRD)
            def _():
                rbase = ci * RD
                pltpu.sync_copy(acc_sh.at[pl.ds(rbase, RD)],
                                rows_s[0].at[pl.ds(0, RD)])
                pltpu.sync_copy(own_ref.at[pl.ds(goff + rbase, RD)],
                                rows_s[1].at[pl.ds(0, RD)])
                pltpu.sync_copy(dd_ref.at[pl.ds(rbase, RD)], dv)

                def dgroup(g, _):
                    dgrp = dv[pl.ds(g * L, L)]

                    def drow(i, _):
                        dsp = _splat(dgrp, i)
                        r = g * L + i
                        for f in range(H // L):
                            sl = pl.ds(f * L, L)
                            rows_s[0][r, sl] = (rows_s[0][r, sl]
                                                + dsp * rows_s[1][r, sl])
                        return 0

                    lax.fori_loop(0, L, drow, 0)
                    return 0

                lax.fori_loop(0, RD // L, dgroup, 0)
                pltpu.sync_copy(rows_s[1].at[pl.ds(0, RD)],
                                out_ref.at[pl.ds(rbase, RD),
                                           pl.ds(cid * H, H)])
                pltpu.sync_copy(rows_s[0].at[pl.ds(0, RD)],
                                out_ref.at[pl.ds(rbase, RD),
                                           pl.ds(F + cid * H, H)])
        plsc.subcore_barrier()


def kernel(ui_edge_index, ui_edge_values, d_i_train, d_j_train,
           embed_user, embed_item):
    row = ui_edge_index[0].astype(jnp.int32)
    col = ui_edge_index[1].astype(jnp.int32)
    val = ui_edge_values.astype(jnp.float32)
    pad = EP - E
    rowp = jnp.concatenate([row, jnp.zeros((pad,), jnp.int32)])
    colp = jnp.concatenate([col, jnp.zeros((pad,), jnp.int32)])
    valp = jnp.concatenate([val, jnp.zeros((pad,), jnp.float32)])
    pk = jnp.stack([colp.reshape(NCHT, K),
                    rowp.reshape(NCHT, K),
                    lax.bitcast_convert_type(valp, jnp.int32)
                       .reshape(NCHT, K)], axis=1)
    ustk = jnp.concatenate([embed_user[:, :H], embed_user[:, H:]], axis=0)
    istk = jnp.concatenate([embed_item[:, :H], embed_item[:, H:]], axis=0)

    mesh = plsc.VectorSubcoreMesh(core_axis_name="c", subcore_axis_name="s")
    run = pl.kernel(
        _body,
        out_type=(
            jax.ShapeDtypeStruct((N_ROWS, 2 * F), jnp.float32),
            jax.ShapeDtypeStruct((N_ROWS, 2 * F), jnp.float32),
        ),
        mesh=mesh,
        scratch_types=[
            [pltpu.VMEM((3, K), jnp.int32) for _ in range(NB)],
            [pltpu.VMEM((K, H), jnp.float32) for _ in range(NB)],
            pltpu.VMEM((RD,), jnp.float32),
            pltpu.VMEM_SHARED((N_ROWS, H), jnp.float32),
            [pltpu.SemaphoreType.DMA for _ in range(NB)],
            [pltpu.SemaphoreType.DMA for _ in range(NB)],
            [pltpu.SemaphoreType.DMA for _ in range(NB)],
        ],
    )
    return run(pk, d_i_train.astype(jnp.float32),
               d_j_train.astype(jnp.float32), ustk, istk)


# speedup vs baseline: 1.0012x; 1.0012x over previous
import jax
import jax.numpy as jnp
from jax import lax
from jax.experimental import pallas as pl
from jax.experimental.pallas import tpu as pltpu
from jax.experimental.pallas import tpu_sc as plsc

N_ROWS = 10000
F = 256
H = 128
E = 160000
NC = 2
NS = 16
L = 16
K = 80
NB = 4
EP = 163840
PER_SUB = EP // NS
NCH = PER_SUB // K
NCHT = EP // K
RD = 80
NRD = N_ROWS // RD
RDPS = -(-NRD // NS)


def _body(pk_ref, di_ref, dj_ref, ustk_ref, istk_ref,
          out_u_ref, out_i_ref,
          pk_s, rows_s, dv, acc_sh, sem_m, sem_g, sem_s):
    cid = lax.axis_index("c")
    sid = lax.axis_index("s")
    goff = cid * N_ROWS
    g0 = sid * NCH
    zero16 = jnp.zeros((L,), jnp.float32)

    def _splat(vec, lane):
        return vec.at[jnp.full((L,), lane, dtype=jnp.int32)].get(
            mode="promise_in_bounds")

    for d in range(2):
        gc, sc = (0, 1) if d == 0 else (1, 0)
        table_ref = istk_ref if d == 0 else ustk_ref
        own_ref = ustk_ref if d == 0 else istk_ref
        dd_ref = di_ref if d == 0 else dj_ref
        out_ref = out_u_ref if d == 0 else out_i_ref

        def _fix(pk):
            for g in range(K // L):
                sl = pl.ds(g * L, L)
                pk[gc, sl] = pk[gc, sl] + goff

        def _scale(rows, pk):
            def egroup(g, _):
                vg = lax.bitcast_convert_type(
                    pk[2, pl.ds(g * L, L)], jnp.float32)

                def erow(i, _):
                    vsp = _splat(vg, i)
                    e = g * L + i
                    for f in range(H // L):
                        sl = pl.ds(f * L, L)
                        rows[e, sl] = rows[e, sl] * vsp
                    return 0

                lax.fori_loop(0, L, erow, 0, unroll=4)
                return 0

            lax.fori_loop(0, K // L, egroup, 0)

        def zrow(r, _):
            for f in range(H // L):
                rows_s[0][r, pl.ds(f * L, L)] = zero16
            return 0

        lax.fori_loop(0, RD, zrow, 0)
        for j in range(RDPS):
            ci = sid + j * NS

            @pl.when(ci < NRD)
            def _():
                pltpu.sync_copy(rows_s[0].at[pl.ds(0, RD)],
                                acc_sh.at[pl.ds(ci * RD, RD)])
        plsc.subcore_barrier()

        def _meta_start(c, s):
            pltpu.async_copy(pk_ref.at[g0 + c], pk_s[s], sem_m[s])

        def _meta_wait(c, s):
            pltpu.make_async_copy(pk_ref.at[g0 + c], pk_s[s],
                                  sem_m[s]).wait()

        def _gather_start(s):
            pltpu.async_copy(table_ref.at[pk_s[s].at[gc]], rows_s[s],
                             sem_g[s])

        def _gather_wait(s):
            pltpu.make_async_copy(table_ref.at[pk_s[s].at[gc]], rows_s[s],
                                  sem_g[s]).wait()

        def _scat_start(s):
            pltpu.async_copy(rows_s[s], acc_sh.at[pk_s[s].at[sc]],
                             sem_s[s], add=True)

        def _scat_wait(s):
            pltpu.make_async_copy(rows_s[s], acc_sh.at[pk_s[s].at[sc]],
                                  sem_s[s]).wait()


        def quad(t, _):
            for j in range(NB):
                c = NB * t + j
                s = j

                @pl.when(c + 1 < NCH)
                def _():
                    _meta_wait(c + 1, (s + 1) % NB)
                    _fix(pk_s[(s + 1) % NB])

                @pl.when(c >= 2)
                def _():
                    _scat_wait((s + 2) % NB)

                @pl.when(c + 2 < NCH)
                def _():
                    _meta_start(c + 2, (s + 2) % NB)
                _gather_wait(s)

                @pl.when(c + 1 < NCH)
                def _():
                    _gather_start((s + 1) % NB)
                _scale(rows_s[s], pk_s[s])
                _scat_start(s)
            return 0

        _meta_start(0, 0)
        _meta_start(1, 1)
        _meta_wait(0, 0)
        _fix(pk_s[0])
        _gather_start(0)
        lax.fori_loop(0, NCH // NB, quad, 0)
        _scat_wait((NCH - 2) % NB)
        _scat_wait((NCH - 1) % NB)
        plsc.subcore_barrier()

        for j in range(RDPS):
            ci = sid + j * NS

            @pl.when(ci < NRD)
            def _():
                rbase = ci * RD
                pltpu.sync_copy(acc_sh.at[pl.ds(rbase, RD)],
                                rows_s[0].at[pl.ds(0, RD)])
                pltpu.sync_copy(own_ref.at[pl.ds(goff + rbase, RD)],
                                rows_s[1].at[pl.ds(0, RD)])
                pltpu.sync_copy(dd_ref.at[pl.ds(rbase, RD)], dv)

                def dgroup(g, _):
                    dgrp = dv[pl.ds(g * L, L)]

                    def drow(i, _):
                        dsp = _splat(dgrp, i)
                        r = g * L + i
                        for f in range(H // L):
                            sl = pl.ds(f * L, L)
                            rows_s[0][r, sl] = (rows_s[0][r, sl]
                                                + dsp * rows_s[1][r, sl])
                        return 0

                    lax.fori_loop(0, L, drow, 0)
                    return 0

                lax.fori_loop(0, RD // L, dgroup, 0)
                pltpu.sync_copy(rows_s[1].at[pl.ds(0, RD)],
                                out_ref.at[pl.ds(rbase, RD),
                                           pl.ds(cid * H, H)])
                pltpu.sync_copy(rows_s[0].at[pl.ds(0, RD)],
                                out_ref.at[pl.ds(rbase, RD),
                                           pl.ds(F + cid * H, H)])
        plsc.subcore_barrier()


def kernel(ui_edge_index, ui_edge_values, d_i_train, d_j_train,
           embed_user, embed_item):
    row = ui_edge_index[0].astype(jnp.int32)
    col = ui_edge_index[1].astype(jnp.int32)
    val = ui_edge_values.astype(jnp.float32)
    pad = EP - E
    rowp = jnp.concatenate([row, jnp.zeros((pad,), jnp.int32)])
    colp = jnp.concatenate([col, jnp.zeros((pad,), jnp.int32)])
    valp = jnp.concatenate([val, jnp.zeros((pad,), jnp.float32)])
    pk = jnp.stack([colp.reshape(NCHT, K),
                    rowp.reshape(NCHT, K),
                    lax.bitcast_convert_type(valp, jnp.int32)
                       .reshape(NCHT, K)], axis=1)
    ustk = jnp.concatenate([embed_user[:, :H], embed_user[:, H:]], axis=0)
    istk = jnp.concatenate([embed_item[:, :H], embed_item[:, H:]], axis=0)

    mesh = plsc.VectorSubcoreMesh(core_axis_name="c", subcore_axis_name="s")
    run = pl.kernel(
        _body,
        out_type=(
            jax.ShapeDtypeStruct((N_ROWS, 2 * F), jnp.float32),
            jax.ShapeDtypeStruct((N_ROWS, 2 * F), jnp.float32),
        ),
        mesh=mesh,
        scratch_types=[
            [pltpu.VMEM((3, K), jnp.int32) for _ in range(NB)],
            [pltpu.VMEM((K, H), jnp.float32) for _ in range(NB)],
            pltpu.VMEM((RD,), jnp.float32),
            pltpu.VMEM_SHARED((N_ROWS, H), jnp.float32),
            [pltpu.SemaphoreType.DMA for _ in range(NB)],
            [pltpu.SemaphoreType.DMA for _ in range(NB)],
            [pltpu.SemaphoreType.DMA for _ in range(NB)],
        ],
    )
    return run(pk, d_i_train.astype(jnp.float32),
               d_j_train.astype(jnp.float32), ustk, istk)


# speedup vs baseline: 1.0279x; 1.0267x over previous
import jax
import jax.numpy as jnp
from jax import lax
from jax.experimental import pallas as pl
from jax.experimental.pallas import tpu as pltpu
from jax.experimental.pallas import tpu_sc as plsc

N_ROWS = 10000
F = 256
H = 128
E = 160000
NC = 2
NS = 16
L = 16
K = 80
NB = 4
EP = 163840
PER_SUB = EP // NS
NCH = PER_SUB // K
NCHT = EP // K
RD = 80
NRD = N_ROWS // RD
RDPS = -(-NRD // NS)


def _body(pk_ref, di_ref, dj_ref, ustk_ref, istk_ref,
          out_u_ref, out_i_ref,
          pk_s, rows_s, dv, acc_sh, sem_m, sem_g, sem_s):
    cid = lax.axis_index("c")
    sid = lax.axis_index("s")
    goff = cid * N_ROWS
    g0 = sid * NCH
    zero16 = jnp.zeros((L,), jnp.float32)

    def _splat(vec, lane):
        return vec.at[jnp.full((L,), lane, dtype=jnp.int32)].get(
            mode="promise_in_bounds")

    for d in range(2):
        gc, sc = (0, 1) if d == 0 else (1, 0)
        table_ref = istk_ref if d == 0 else ustk_ref
        own_ref = ustk_ref if d == 0 else istk_ref
        dd_ref = di_ref if d == 0 else dj_ref
        out_ref = out_u_ref if d == 0 else out_i_ref

        def _fix(pk):
            for g in range(K // L):
                sl = pl.ds(g * L, L)
                pk[gc, sl] = pk[gc, sl] + goff

        def _scale(rows, pk):
            def egroup(g, _):
                vg = lax.bitcast_convert_type(
                    pk[2, pl.ds(g * L, L)], jnp.float32)

                def erow(i, _):
                    vsp = _splat(vg, i)
                    e = g * L + i
                    for f in range(H // L):
                        sl = pl.ds(f * L, L)
                        rows[e, sl] = rows[e, sl] * vsp
                    return 0

                lax.fori_loop(0, L, erow, 0, unroll=4)
                return 0

            lax.fori_loop(0, K // L, egroup, 0)

        def zrow(r, _):
            for f in range(H // L):
                rows_s[0][r, pl.ds(f * L, L)] = zero16
            return 0

        lax.fori_loop(0, RD, zrow, 0)
        for j in range(RDPS):
            ci = sid + j * NS

            @pl.when(ci < NRD)
            def _():
                pltpu.sync_copy(rows_s[0].at[pl.ds(0, RD)],
                                acc_sh.at[pl.ds(ci * RD, RD)])
        plsc.subcore_barrier()

        def _meta_start(c, s):
            pltpu.async_copy(pk_ref.at[g0 + c], pk_s[s], sem_m[s])

        def _meta_wait(c, s):
            pltpu.make_async_copy(pk_ref.at[g0 + c], pk_s[s],
                                  sem_m[s]).wait()

        def _gather_start(s):
            pltpu.async_copy(table_ref.at[pk_s[s].at[gc]], rows_s[s],
                             sem_g[s])

        def _gather_wait(s):
            pltpu.make_async_copy(table_ref.at[pk_s[s].at[gc]], rows_s[s],
                                  sem_g[s]).wait()

        def _scat_start(s):
            pltpu.async_copy(rows_s[s], acc_sh.at[pk_s[s].at[sc]],
                             sem_s[s], add=True)

        def _scat_wait(s):
            pltpu.make_async_copy(rows_s[s], acc_sh.at[pk_s[s].at[sc]],
                                  sem_s[s]).wait()


        def quad(t, _):
            for j in range(NB):
                c = NB * t + j
                s = j

                @pl.when(c + 1 < NCH)
                def _():
                    _meta_wait(c + 1, (s + 1) % NB)
                    _fix(pk_s[(s + 1) % NB])

                @pl.when(c >= 2)
                def _():
                    _scat_wait((s + 2) % NB)

                @pl.when(c + 2 < NCH)
                def _():
                    _meta_start(c + 2, (s + 2) % NB)
                _gather_wait(s)

                @pl.when(c + 1 < NCH)
                def _():
                    _gather_start((s + 1) % NB)
                _scale(rows_s[s], pk_s[s])
                _scat_start(s)
            return 0

        _meta_start(0, 0)
        _meta_start(1, 1)
        _meta_wait(0, 0)
        _fix(pk_s[0])
        _gather_start(0)
        lax.fori_loop(0, NCH // NB, quad, 0)
        _scat_wait((NCH - 2) % NB)
        _scat_wait((NCH - 1) % NB)
        plsc.subcore_barrier()

        def _dr_read_start(ci, p):
            rbase = ci * RD
            pltpu.async_copy(acc_sh.at[pl.ds(rbase, RD)], rows_s[p],
                             sem_s[p])
            pltpu.async_copy(own_ref.at[pl.ds(goff + rbase, RD)],
                             rows_s[p + 2], sem_g[p])
            pltpu.async_copy(dd_ref.at[pl.ds(rbase, RD)], dv[p], sem_m[p])

        def _dr_read_wait(ci, p):
            rbase = ci * RD
            pltpu.make_async_copy(acc_sh.at[pl.ds(rbase, RD)], rows_s[p],
                                  sem_s[p]).wait()
            pltpu.make_async_copy(own_ref.at[pl.ds(goff + rbase, RD)],
                                  rows_s[p + 2], sem_g[p]).wait()
            pltpu.make_async_copy(dd_ref.at[pl.ds(rbase, RD)], dv[p],
                                  sem_m[p]).wait()

        def _dr_write_start(ci, p):
            rbase = ci * RD
            pltpu.async_copy(rows_s[p + 2],
                             out_ref.at[pl.ds(rbase, RD),
                                        pl.ds(cid * H, H)],
                             sem_g[p + 2])
            pltpu.async_copy(rows_s[p],
                             out_ref.at[pl.ds(rbase, RD),
                                        pl.ds(F + cid * H, H)],
                             sem_s[p + 2])

        def _dr_write_wait(ci, p):
            rbase = ci * RD
            pltpu.make_async_copy(rows_s[p + 2],
                                  out_ref.at[pl.ds(rbase, RD),
                                             pl.ds(cid * H, H)],
                                  sem_g[p + 2]).wait()
            pltpu.make_async_copy(rows_s[p],
                                  out_ref.at[pl.ds(rbase, RD),
                                             pl.ds(F + cid * H, H)],
                                  sem_s[p + 2]).wait()

        _dr_read_start(sid, 0)
        for j in range(RDPS):
            ci = sid + j * NS
            p = j % 2

            @pl.when(ci < NRD)
            def _():
                _dr_read_wait(ci, p)

                def dgroup(g, _):
                    dgrp = dv[p][pl.ds(g * L, L)]

                    def drow(i, _):
                        dsp = _splat(dgrp, i)
                        r = g * L + i
                        for f in range(H // L):
                            sl = pl.ds(f * L, L)
                            rows_s[p][r, sl] = (rows_s[p][r, sl]
                                                + dsp * rows_s[p + 2][r, sl])
                        return 0

                    lax.fori_loop(0, L, drow, 0)
                    return 0

                lax.fori_loop(0, RD // L, dgroup, 0)
                _dr_write_start(ci, p)
                if j + 1 < RDPS:
                    @pl.when(ci + NS < NRD)
                    def _():
                        if j >= 1:
                            _dr_write_wait(ci - NS, (j + 1) % 2)
                        _dr_read_start(ci + NS, (j + 1) % 2)
        for j in range(RDPS):
            ci = sid + j * NS
            if j + 2 < RDPS:
                @pl.when((ci < NRD) & (ci + 2 * NS >= NRD))
                def _():
                    _dr_write_wait(ci, j % 2)
            else:
                @pl.when(ci < NRD)
                def _():
                    _dr_write_wait(ci, j % 2)
        plsc.subcore_barrier()


def kernel(ui_edge_index, ui_edge_values, d_i_train, d_j_train,
           embed_user, embed_item):
    row = ui_edge_index[0].astype(jnp.int32)
    col = ui_edge_index[1].astype(jnp.int32)
    val = ui_edge_values.astype(jnp.float32)
    pad = EP - E
    rowp = jnp.concatenate([row, jnp.zeros((pad,), jnp.int32)])
    colp = jnp.concatenate([col, jnp.zeros((pad,), jnp.int32)])
    valp = jnp.concatenate([val, jnp.zeros((pad,), jnp.float32)])
    pk = jnp.stack([colp.reshape(NCHT, K),
                    rowp.reshape(NCHT, K),
                    lax.bitcast_convert_type(valp, jnp.int32)
                       .reshape(NCHT, K)], axis=1)
    ustk = jnp.concatenate([embed_user[:, :H], embed_user[:, H:]], axis=0)
    istk = jnp.concatenate([embed_item[:, :H], embed_item[:, H:]], axis=0)

    mesh = plsc.VectorSubcoreMesh(core_axis_name="c", subcore_axis_name="s")
    run = pl.kernel(
        _body,
        out_type=(
            jax.ShapeDtypeStruct((N_ROWS, 2 * F), jnp.float32),
            jax.ShapeDtypeStruct((N_ROWS, 2 * F), jnp.float32),
        ),
        mesh=mesh,
        scratch_types=[
            [pltpu.VMEM((3, K), jnp.int32) for _ in range(NB)],
            [pltpu.VMEM((K, H), jnp.float32) for _ in range(NB)],
            [pltpu.VMEM((RD,), jnp.float32) for _ in range(2)],
            pltpu.VMEM_SHARED((N_ROWS, H), jnp.float32),
            [pltpu.SemaphoreType.DMA for _ in range(NB)],
            [pltpu.SemaphoreType.DMA for _ in range(NB)],
            [pltpu.SemaphoreType.DMA for _ in range(NB)],
        ],
    )
    return run(pk, d_i_train.astype(jnp.float32),
               d_j_train.astype(jnp.float32), ustk, istk)
